# baseline (speedup 1.0000x reference)
import functools

import jax
import jax.numpy as jnp
from jax import lax
from jax.experimental import pallas as pl
from jax.experimental.pallas import tpu as pltpu
from jax.experimental.pallas import tpu_sc as plsc

BATCH = 16384
DIM = 64
LANES = 16
NUM_CORES = 2
NUM_SUBCORES = 16
NUM_WORKERS = NUM_CORES * NUM_SUBCORES
B_PER_W = BATCH // NUM_WORKERS
IDX_CHUNK = 128


def _make_kernel():
    mesh = plsc.VectorSubcoreMesh(core_axis_name="c", subcore_axis_name="s")

    @functools.partial(
        pl.kernel,
        mesh=mesh,
        out_type=jax.ShapeDtypeStruct((BATCH,), jnp.float32),
        compiler_params=pltpu.CompilerParams(use_tc_tiling_on_sc=False),
        scratch_types=[
            pltpu.VMEM((B_PER_W,), jnp.int32),
            pltpu.VMEM((B_PER_W,), jnp.int32),
            pltpu.VMEM((B_PER_W,), jnp.int32),
            pltpu.VMEM((B_PER_W, DIM), jnp.float32),
            pltpu.VMEM((B_PER_W, DIM), jnp.float32),
            pltpu.VMEM((B_PER_W, DIM), jnp.float32),
            pltpu.VMEM((B_PER_W,), jnp.float32),
            pltpu.SemaphoreType.DMA,
        ],
    )
    def distmult(head_hbm, rel_hbm, tail_hbm, ent_hbm, relemb_hbm, out_hbm,
                 hidx, ridx, tidx, hrows, rrows, trows, scores, sem):
        wid = lax.axis_index("s") * NUM_CORES + lax.axis_index("c")
        base = wid * B_PER_W

        pltpu.sync_copy(head_hbm.at[pl.ds(base, B_PER_W)], hidx)
        pltpu.sync_copy(rel_hbm.at[pl.ds(base, B_PER_W)], ridx)
        pltpu.sync_copy(tail_hbm.at[pl.ds(base, B_PER_W)], tidx)

        copies = []
        for c in range(B_PER_W // IDX_CHUNK):
            sl = pl.ds(c * IDX_CHUNK, IDX_CHUNK)
            copies.append(pltpu.async_copy(ent_hbm.at[hidx.at[sl]], hrows.at[sl], sem))
            copies.append(pltpu.async_copy(relemb_hbm.at[ridx.at[sl]], rrows.at[sl], sem))
            copies.append(pltpu.async_copy(ent_hbm.at[tidx.at[sl]], trows.at[sl], sem))
        for cp in copies:
            cp.wait()

        lane = lax.iota(jnp.int32, LANES)

        dnums = lax.GatherDimensionNumbers(
            offset_dims=(), collapsed_slice_dims=(0,), start_index_map=(0,))

        def shuffle(v, idx):
            return lax.gather(v, idx[:, None], dnums, slice_sizes=(1,),
                              mode=lax.GatherScatterMode.PROMISE_IN_BOUNDS)

        def lane_sum(v):
            for sh in (8, 4, 2, 1):
                v = v + shuffle(v, lane ^ sh)
            return v

        def group(g, carry):
            svec = jnp.zeros((LANES,), jnp.float32)
            for j in range(LANES):
                b = g * LANES + j
                acc = hrows[b, pl.ds(0, LANES)] * rrows[b, pl.ds(0, LANES)] * trows[b, pl.ds(0, LANES)]
                for dc in range(1, DIM // LANES):
                    sl2 = pl.ds(dc * LANES, LANES)
                    acc = acc + hrows[b, sl2] * rrows[b, sl2] * trows[b, sl2]
                svec = jnp.where(lane == j, -lane_sum(acc), svec)
            scores[pl.ds(g * LANES, LANES)] = svec
            return carry

        lax.fori_loop(0, B_PER_W // LANES, group, 0)

        pltpu.sync_copy(scores, out_hbm.at[pl.ds(base, B_PER_W)])

    return distmult


_distmult = _make_kernel()


@jax.jit
def kernel(head, rel, tail, entity_emb, relation_emb):
    return _distmult(head, rel, tail, entity_emb, relation_emb)
